# Initial kernel scaffold; baseline (speedup 1.0000x reference)
import functools

import jax
import jax.numpy as jnp
from jax import lax
from jax.experimental import pallas as pl
from jax.experimental.pallas import tpu as pltpu
from jax.experimental.pallas import tpu_sc as plsc

N = 100000
E = 1600000
D = 4
BIG = 32
HID = 2
G = 128

NPAD = 102400
EPAD = 1638400
W = 128
TILE_E = 1024
TILE_N = 1024
NSUB = 16
NSTRIPE = NPAD // NSUB

_SC_MESH = functools.partial(
    plsc.VectorSubcoreMesh, core_axis_name="c", subcore_axis_name="s")


def _stats_body(x_ref, o_ref):
    i = pl.program_id(0)

    @pl.when(i == 0)
    def _():
        o_ref[...] = jnp.zeros_like(o_ref)

    xb = x_ref[...]
    o_ref[0:1, :] += jnp.sum(xb, axis=0, keepdims=True)
    o_ref[1:2, :] += jnp.sum(xb * xb, axis=0, keepdims=True)


def _stats(xpad):
    return pl.pallas_call(
        _stats_body,
        grid=(NPAD // TILE_N,),
        in_specs=[pl.BlockSpec((TILE_N, D), lambda i: (i, 0))],
        out_specs=pl.BlockSpec((2, D), lambda i: (0, 0)),
        out_shape=jax.ShapeDtypeStruct((2, D), jnp.float32),
    )(xpad)


def _sc_gather(table, dsti, srci):
    @functools.partial(
        pl.kernel,
        out_type=(jax.ShapeDtypeStruct((EPAD, D), jnp.float32),
                  jax.ShapeDtypeStruct((EPAD, D), jnp.float32)),
        mesh=_SC_MESH(),
        scratch_types=[pltpu.VMEM_SHARED((NPAD, D), jnp.float32)],
    )
    def k(table_hbm, dsti_hbm, srci_hbm, oi_hbm, oj_hbm, stab):
        sid = lax.axis_index("s")
        sl = pl.ds(sid * NSTRIPE, NSTRIPE)
        pltpu.sync_copy(table_hbm.at[sl], stab.at[sl])
        plsc.subcore_barrier()

        def body(di_v, si_v, oi_v, oj_v):
            pltpu.sync_copy(stab.at[di_v.at[0]], oi_v)
            pltpu.sync_copy(stab.at[si_v.at[0]], oj_v)

        pltpu.emit_pipeline(
            body,
            grid=(EPAD // W,),
            in_specs=[pl.BlockSpec((1, W), lambda i: (0, i)),
                      pl.BlockSpec((1, W), lambda i: (0, i))],
            out_specs=[pl.BlockSpec((W, D), lambda i: (i, 0)),
                       pl.BlockSpec((W, D), lambda i: (i, 0))],
            core_axis_name=("c", "s"),
            dimension_semantics=(pltpu.PARALLEL,),
        )(dsti_hbm, srci_hbm, oi_hbm, oj_hbm)

    return k(table, dsti, srci)


def _sc_scatter(m, dsti, zeros_tab):
    @functools.partial(
        pl.kernel,
        out_type=jax.ShapeDtypeStruct((2, NPAD, D), jnp.float32),
        mesh=_SC_MESH(),
        scratch_types=[pltpu.VMEM_SHARED((NPAD, D), jnp.float32)],
    )
    def k(m_hbm, dsti_hbm, z_hbm, out_hbm, acc):
        cid = lax.axis_index("c")
        sid = lax.axis_index("s")
        sl = pl.ds(sid * NSTRIPE, NSTRIPE)
        pltpu.sync_copy(z_hbm.at[sl], acc.at[sl])
        plsc.subcore_barrier()

        def body(m_v, di_v):
            pltpu.sync_copy(m_v, acc.at[di_v.at[0]], add=True)

        pltpu.emit_pipeline(
            body,
            grid=(EPAD // W,),
            in_specs=[pl.BlockSpec((W, D), lambda i: (i, 0)),
                      pl.BlockSpec((1, W), lambda i: (0, i))],
            out_specs=[],
            core_axis_name=("c", "s"),
            dimension_semantics=(pltpu.PARALLEL,),
        )(m_hbm, dsti_hbm)
        plsc.subcore_barrier()
        pltpu.sync_copy(acc.at[sl], out_hbm.at[cid].at[sl])

    return k(m, dsti, zeros_tab)


def _mlp_body(final_relu, xi_ref, xj_ref, wa_ref, wb_ref, c1_ref,
              w2_ref, b2_ref, w3_ref, b3_ref, o_ref):
    h = (jnp.dot(xi_ref[...], wa_ref[...], precision="highest")
         + jnp.dot(xj_ref[...], wb_ref[...], precision="highest")
         + c1_ref[...])
    h = jnp.maximum(h, 0.0)
    h = jnp.dot(h, w2_ref[...], precision="highest") + b2_ref[...]
    h = jnp.maximum(h, 0.0)
    h = jnp.dot(h, w3_ref[...], precision="highest") + b3_ref[...]
    if final_relu:
        h = jnp.maximum(h, 0.0)
    o_ref[...] = h


def _edge_mlp(xi, xj, wa, wb, c1, w2, b2, w3, b3, final_relu):
    full = lambda r, c: pl.BlockSpec((r, c), lambda i: (0, 0))
    return pl.pallas_call(
        functools.partial(_mlp_body, final_relu),
        grid=(EPAD // TILE_E,),
        in_specs=[pl.BlockSpec((TILE_E, D), lambda i: (i, 0)),
                  pl.BlockSpec((TILE_E, D), lambda i: (i, 0)),
                  full(D, BIG), full(D, BIG), full(1, BIG),
                  full(BIG, BIG), full(1, BIG),
                  full(BIG, D), full(1, D)],
        out_specs=pl.BlockSpec((TILE_E, D), lambda i: (i, 0)),
        out_shape=jax.ShapeDtypeStruct((EPAD, D), jnp.float32),
    )(xi, xj, wa, wb, c1, w2, b2, w3, b3)


def _combine_body(p_ref, htab_ref, cnt_ref):
    s = p_ref[0] + p_ref[1]
    cnt = s[:, 2:3]
    denom = jnp.maximum(cnt, 1.0)
    h01 = s[:, 0:2] / denom
    htab_ref[...] = jnp.concatenate(
        [h01, jnp.zeros((TILE_N, 2), jnp.float32)], axis=1)
    cnt_ref[...] = cnt


def _combine(penc):
    return pl.pallas_call(
        _combine_body,
        grid=(NPAD // TILE_N,),
        in_specs=[pl.BlockSpec((2, TILE_N, D), lambda i: (0, i, 0))],
        out_specs=(pl.BlockSpec((TILE_N, D), lambda i: (i, 0)),
                   pl.BlockSpec((TILE_N, 1), lambda i: (i, 0))),
        out_shape=(jax.ShapeDtypeStruct((NPAD, D), jnp.float32),
                   jax.ShapeDtypeStruct((NPAD, 1), jnp.float32)),
    )(penc)


def _final_body(p_ref, cnt_ref, x_ref, b_ref, out_ref, emd_ref,
                acc_sum, acc_cnt):
    i = pl.program_id(0)

    @pl.when(i == 0)
    def _():
        acc_sum[...] = jnp.zeros_like(acc_sum)
        acc_cnt[...] = jnp.zeros_like(acc_cnt)

    s = p_ref[0] + p_ref[1]
    denom = jnp.maximum(cnt_ref[...], 1.0)
    out = s / denom
    out_ref[...] = out
    d = out - x_ref[...]
    diff = jnp.sum(d * d, axis=1, keepdims=True)
    b = b_ref[0]
    onehot = (b.reshape(TILE_N, 1) ==
              lax.broadcasted_iota(jnp.int32, (1, G), 1)).astype(jnp.float32)
    acc_sum[...] += jnp.sum(onehot * diff, axis=0, keepdims=True)
    acc_cnt[...] += jnp.sum(onehot, axis=0, keepdims=True)

    @pl.when(i == pl.num_programs(0) - 1)
    def _():
        emd_ref[...] = acc_sum[...] / jnp.maximum(acc_cnt[...], 1.0)


def _final(pdec, cnt, xpad, batch3):
    return pl.pallas_call(
        _final_body,
        grid=(NPAD // TILE_N,),
        in_specs=[pl.BlockSpec((2, TILE_N, D), lambda i: (0, i, 0)),
                  pl.BlockSpec((TILE_N, 1), lambda i: (i, 0)),
                  pl.BlockSpec((TILE_N, D), lambda i: (i, 0)),
                  pl.BlockSpec((1, 1, TILE_N), lambda i: (i, 0, 0))],
        out_specs=(pl.BlockSpec((TILE_N, D), lambda i: (i, 0)),
                   pl.BlockSpec((1, G), lambda i: (0, 0))),
        out_shape=(jax.ShapeDtypeStruct((NPAD, D), jnp.float32),
                   jax.ShapeDtypeStruct((1, G), jnp.float32)),
        scratch_shapes=[pltpu.VMEM((1, G), jnp.float32),
                        pltpu.VMEM((1, G), jnp.float32)],
    )(pdec, cnt, xpad, batch3)


def kernel(x, edge_index, batch, bn_gamma, bn_beta,
           enc_w1, enc_b1, enc_w2, enc_b2, enc_w3, enc_b3,
           dec_w1, dec_b1, dec_w2, dec_b2, dec_w3, dec_b3):
    f32 = jnp.float32
    xpad = jnp.pad(x, ((0, NPAD - N), (0, 0)))
    ei = jnp.pad(edge_index.astype(jnp.int32), ((0, 0), (0, EPAD - E)),
                 constant_values=NPAD - 1)
    srci = ei[0:1]
    dsti = ei[1:2]
    batch3 = jnp.pad(batch.astype(jnp.int32), (0, NPAD - N),
                     constant_values=-1).reshape(NPAD // TILE_N, 1, TILE_N)
    zeros_tab = jnp.zeros((NPAD, D), f32)

    st = _stats(xpad)
    mean = st[0] / N
    var = st[1] / N - mean * mean
    scale = bn_gamma / jnp.sqrt(var + 1e-5)
    shift = bn_beta - mean * scale

    e_wa = scale[:, None] * (enc_w1[:D] - enc_w1[D:])
    e_wb = scale[:, None] * enc_w1[D:]
    e_c1 = (enc_b1 + shift @ enc_w1[:D]).reshape(1, BIG)
    e_w3 = jnp.concatenate([enc_w3, jnp.zeros((BIG, 2), f32)], axis=1)
    e_b3 = jnp.concatenate([enc_b3, jnp.array([1.0, 0.0], f32)]).reshape(1, D)

    d_wa = jnp.pad(dec_w1[:HID] - dec_w1[HID:], ((0, D - HID), (0, 0)))
    d_wb = jnp.pad(dec_w1[HID:], ((0, D - HID), (0, 0)))
    d_c1 = dec_b1.reshape(1, BIG)

    xi, xj = _sc_gather(xpad, dsti, srci)
    m1 = _edge_mlp(xi, xj, e_wa, e_wb, e_c1, enc_w2,
                   enc_b2.reshape(1, BIG), e_w3, e_b3, final_relu=True)
    penc = _sc_scatter(m1, dsti, zeros_tab)
    htab, cnt = _combine(penc)

    hi, hj = _sc_gather(htab, dsti, srci)
    m2 = _edge_mlp(hi, hj, d_wa, d_wb, d_c1, dec_w2,
                   dec_b2.reshape(1, BIG), dec_w3, dec_b3.reshape(1, D),
                   final_relu=False)
    pdec = _sc_scatter(m2, dsti, zeros_tab)

    outp, emd2 = _final(pdec, cnt, xpad, batch3)
    return outp[:N], emd2[0]


# speedup vs baseline: 12.9217x; 12.9217x over previous
import functools

import jax
import jax.numpy as jnp
from jax import lax
from jax.experimental import pallas as pl
from jax.experimental.pallas import tpu as pltpu
from jax.experimental.pallas import tpu_sc as plsc

N = 100000
E = 1600000
D = 4
BIG = 32
HID = 2
G = 128

NPAD = 102400
EPAD = 1638400
W = 128
TILE_E = 1024
TILE_N = 1024
NSUB = 16
STRIPE = NPAD // NSUB

_SC_MESH = functools.partial(
    plsc.VectorSubcoreMesh, core_axis_name="c", subcore_axis_name="s")


def _stats_body(x_ref, o_ref):
    @pl.when(pl.program_id(0) == 0)
    def _():
        o_ref[...] = jnp.zeros_like(o_ref)

    xb = x_ref[...]
    o_ref[:, 0:1] += jnp.sum(xb, axis=1, keepdims=True)
    o_ref[:, 1:2] += jnp.sum(xb * xb, axis=1, keepdims=True)


def _stats(xT):
    return pl.pallas_call(
        _stats_body,
        grid=(NPAD // TILE_N,),
        in_specs=[pl.BlockSpec((D, TILE_N), lambda i: (0, i))],
        out_specs=pl.BlockSpec((D, 2), lambda i: (0, 0)),
        out_shape=jax.ShapeDtypeStruct((D, 2), jnp.float32),
    )(xT)


def _sc_gather(tables, ei):
    nt = len(tables)

    @functools.partial(
        pl.kernel,
        out_type=jax.ShapeDtypeStruct((2 * nt, EPAD), jnp.float32),
        mesh=_SC_MESH(),
    )
    def k(*refs):
        t_hbm = refs[:nt]
        ei_hbm = refs[nt]
        o_hbm = refs[nt + 1]

        def body(di_v, si_v, o_v):
            for c in range(nt):
                pltpu.sync_copy(t_hbm[c].at[di_v.at[0]], o_v.at[c])
                pltpu.sync_copy(t_hbm[c].at[si_v.at[0]], o_v.at[nt + c])

        pltpu.emit_pipeline(
            body,
            grid=(EPAD // W,),
            in_specs=[pl.BlockSpec((1, W), lambda i: (1, i)),
                      pl.BlockSpec((1, W), lambda i: (0, i))],
            out_specs=[pl.BlockSpec((2 * nt, W), lambda i: (0, i))],
            core_axis_name=("c", "s"),
            dimension_semantics=(pltpu.PARALLEL,),
        )(ei_hbm, ei_hbm, o_hbm)

    return k(*tables, ei)


def _sc_scatter(m, ei, npl):
    @functools.partial(
        pl.kernel,
        out_type=jax.ShapeDtypeStruct((2 * npl * NPAD,), jnp.float32),
        mesh=_SC_MESH(),
        scratch_types=[pltpu.VMEM_SHARED((NPAD,), jnp.float32)
                       for _ in range(npl)]
        + [pltpu.VMEM((STRIPE,), jnp.float32)],
    )
    def k(m_hbm, ei_hbm, o_hbm, *scr):
        acc = scr[:npl]
        vbuf = scr[-1]
        cid = lax.axis_index("c")
        sid = lax.axis_index("s")
        sl = pl.ds(sid * STRIPE, STRIPE)

        @pl.loop(0, STRIPE, step=16)
        def _(i):
            vbuf[pl.ds(i, 16)] = jnp.zeros((16,), jnp.float32)

        for c in range(npl):
            pltpu.sync_copy(vbuf, acc[c].at[sl])
        plsc.subcore_barrier()

        def body(m_v, di_v):
            for c in range(npl):
                pltpu.sync_copy(m_v.at[c], acc[c].at[di_v.at[0]], add=True)

        pltpu.emit_pipeline(
            body,
            grid=(EPAD // W,),
            in_specs=[pl.BlockSpec((npl, W), lambda i: (0, i)),
                      pl.BlockSpec((1, W), lambda i: (1, i))],
            out_specs=[],
            core_axis_name=("c", "s"),
            dimension_semantics=(pltpu.PARALLEL,),
        )(m_hbm, ei_hbm)
        plsc.subcore_barrier()
        for c in range(npl):
            pltpu.sync_copy(acc[c].at[sl], vbuf)
            pltpu.sync_copy(
                vbuf,
                o_hbm.at[pl.ds((cid * npl + c) * NPAD + sid * STRIPE,
                               STRIPE)])

    return k(m, ei).reshape(2, npl, NPAD)


def _mlp_body(ni, final_relu, x_ref, waT_ref, wbT_ref, c1_ref,
              w2T_ref, b2_ref, w3T_ref, b3_ref, o_ref):
    x = x_ref[...]
    waT = waT_ref[...]
    wbT = wbT_ref[...]
    acc = None
    for c in range(ni):
        t = (waT[:, c:c + 1] * x[c:c + 1, :]
             + wbT[:, c:c + 1] * x[ni + c:ni + c + 1, :])
        acc = t if acc is None else acc + t
    h = jnp.maximum(acc + c1_ref[...], 0.0)
    h = jnp.maximum(
        jnp.dot(w2T_ref[...], h, precision="highest") + b2_ref[...], 0.0)
    h = jnp.dot(w3T_ref[...], h, precision="highest") + b3_ref[...]
    if final_relu:
        h = jnp.maximum(h, 0.0)
    o_ref[...] = h


def _edge_mlp(xe, waT, wbT, c1, w2T, b2, w3T, b3, final_relu):
    ni = waT.shape[1]
    no = w3T.shape[0]
    full = lambda r, c: pl.BlockSpec((r, c), lambda i: (0, 0))
    return pl.pallas_call(
        functools.partial(_mlp_body, ni, final_relu),
        grid=(EPAD // TILE_E,),
        in_specs=[pl.BlockSpec((2 * ni, TILE_E), lambda i: (0, i)),
                  full(BIG, ni), full(BIG, ni), full(BIG, 1),
                  full(BIG, BIG), full(BIG, 1),
                  full(no, BIG), full(no, 1)],
        out_specs=pl.BlockSpec((no, TILE_E), lambda i: (0, i)),
        out_shape=jax.ShapeDtypeStruct((no, EPAD), jnp.float32),
    )(xe, waT, wbT, c1, w2T, b2, w3T, b3)


def _combine_body(p_ref, h_ref, cnt_ref):
    p = p_ref[...]
    s = p[0] + p[1]
    cnt = s[2:3, :]
    denom = jnp.maximum(cnt, 1.0)
    h_ref[...] = s[0:2, :] / denom
    cnt_ref[...] = cnt


def _combine(penc):
    return pl.pallas_call(
        _combine_body,
        grid=(NPAD // TILE_N,),
        in_specs=[pl.BlockSpec((2, 3, TILE_N), lambda i: (0, 0, i))],
        out_specs=(pl.BlockSpec((2, TILE_N), lambda i: (0, i)),
                   pl.BlockSpec((1, TILE_N), lambda i: (0, i))),
        out_shape=(jax.ShapeDtypeStruct((2, NPAD), jnp.float32),
                   jax.ShapeDtypeStruct((1, NPAD), jnp.float32)),
    )(penc)


def _final_body(p_ref, cnt_ref, x_ref, b_ref, out_ref, emd_ref,
                acc_sum, acc_cnt):
    i = pl.program_id(0)

    @pl.when(i == 0)
    def _():
        acc_sum[...] = jnp.zeros_like(acc_sum)
        acc_cnt[...] = jnp.zeros_like(acc_cnt)

    p = p_ref[...]
    s = p[0] + p[1]
    denom = jnp.maximum(cnt_ref[...], 1.0)
    out = s / denom
    out_ref[...] = out
    d = out - x_ref[...]
    diff = jnp.sum(d * d, axis=0, keepdims=True)
    onehot = (lax.broadcasted_iota(jnp.int32, (G, 1), 0) ==
              b_ref[...]).astype(jnp.float32)
    acc_sum[...] += jnp.sum(onehot * diff, axis=1, keepdims=True)
    acc_cnt[...] += jnp.sum(onehot, axis=1, keepdims=True)

    @pl.when(i == pl.num_programs(0) - 1)
    def _():
        emd_ref[...] = acc_sum[...] / jnp.maximum(acc_cnt[...], 1.0)


def _final(pdec, cnt, xT, batch2):
    return pl.pallas_call(
        _final_body,
        grid=(NPAD // TILE_N,),
        in_specs=[pl.BlockSpec((2, D, TILE_N), lambda i: (0, 0, i)),
                  pl.BlockSpec((1, TILE_N), lambda i: (0, i)),
                  pl.BlockSpec((D, TILE_N), lambda i: (0, i)),
                  pl.BlockSpec((1, TILE_N), lambda i: (0, i))],
        out_specs=(pl.BlockSpec((D, TILE_N), lambda i: (0, i)),
                   pl.BlockSpec((G, 1), lambda i: (0, 0))),
        out_shape=(jax.ShapeDtypeStruct((D, NPAD), jnp.float32),
                   jax.ShapeDtypeStruct((G, 1), jnp.float32)),
        scratch_shapes=[pltpu.VMEM((G, 1), jnp.float32),
                        pltpu.VMEM((G, 1), jnp.float32)],
    )(pdec, cnt, xT, batch2)


def kernel(x, edge_index, batch, bn_gamma, bn_beta,
           enc_w1, enc_b1, enc_w2, enc_b2, enc_w3, enc_b3,
           dec_w1, dec_b1, dec_w2, dec_b2, dec_w3, dec_b3):
    f32 = jnp.float32
    xT = jnp.pad(x.T, ((0, 0), (0, NPAD - N)))
    ei = jnp.pad(edge_index.astype(jnp.int32), ((0, 0), (0, EPAD - E)),
                 constant_values=NPAD - 1)
    batch2 = jnp.pad(batch.astype(jnp.int32), (0, NPAD - N),
                     constant_values=-1).reshape(1, NPAD)

    st = _stats(xT)
    mean = st[:, 0] / N
    var = st[:, 1] / N - mean * mean
    scale = bn_gamma / jnp.sqrt(var + 1e-5)
    shift = bn_beta - mean * scale

    e_waT = (scale[:, None] * (enc_w1[:D] - enc_w1[D:])).T
    e_wbT = (scale[:, None] * enc_w1[D:]).T
    e_c1 = (enc_b1 + shift @ enc_w1[:D]).reshape(BIG, 1)
    e_w3T = jnp.concatenate([enc_w3.T, jnp.zeros((1, BIG), f32)])
    e_b3 = jnp.concatenate([enc_b3, jnp.ones((1,), f32)]).reshape(3, 1)

    d_waT = (dec_w1[:HID] - dec_w1[HID:]).T
    d_wbT = dec_w1[HID:].T
    d_c1 = dec_b1.reshape(BIG, 1)

    xe = _sc_gather([xT[0], xT[1], xT[2], xT[3]], ei)
    m1 = _edge_mlp(xe, e_waT, e_wbT, e_c1, enc_w2.T,
                   enc_b2.reshape(BIG, 1), e_w3T, e_b3, final_relu=True)
    penc = _sc_scatter(m1, ei, 3)
    h2d, cnt = _combine(penc)

    xd = _sc_gather([h2d[0], h2d[1]], ei)
    m2 = _edge_mlp(xd, d_waT, d_wbT, d_c1, dec_w2.T,
                   dec_b2.reshape(BIG, 1), dec_w3.T,
                   dec_b3.reshape(D, 1), final_relu=False)
    pdec = _sc_scatter(m2, ei, 4)

    outT, emd = _final(pdec, cnt, xT, batch2)
    return outT[:, :N].T, emd.reshape(G)


# speedup vs baseline: 19.7812x; 1.5309x over previous
import functools

import jax
import jax.numpy as jnp
from jax import lax
from jax.experimental import pallas as pl
from jax.experimental.pallas import tpu as pltpu
from jax.experimental.pallas import tpu_sc as plsc

N = 100000
E = 1600000
D = 4
BIG = 32
HID = 2
G = 128

NPAD = 102400
EPAD = 1638400
W = 128
TILE_E = 1024
TILE_N = 1024
NSUB = 16
STRIPE = NPAD // NSUB

_SC_MESH = functools.partial(
    plsc.VectorSubcoreMesh, core_axis_name="c", subcore_axis_name="s")


def _stats_body(x_ref, o_ref):
    @pl.when(pl.program_id(0) == 0)
    def _():
        o_ref[...] = jnp.zeros_like(o_ref)

    xb = x_ref[...]
    o_ref[:, 0:1] += jnp.sum(xb, axis=1, keepdims=True)
    o_ref[:, 1:2] += jnp.sum(xb * xb, axis=1, keepdims=True)


def _stats(xT):
    return pl.pallas_call(
        _stats_body,
        grid=(NPAD // TILE_N,),
        in_specs=[pl.BlockSpec((D, TILE_N), lambda i: (0, i))],
        out_specs=pl.BlockSpec((D, 2), lambda i: (0, 0)),
        out_shape=jax.ShapeDtypeStruct((D, 2), jnp.float32),
    )(xT)


def _sc_gather(tables, ei):
    nt = len(tables)

    @functools.partial(
        pl.kernel,
        out_type=jax.ShapeDtypeStruct((2 * nt, EPAD), jnp.float32),
        mesh=_SC_MESH(),
        scratch_types=[pltpu.VMEM_SHARED((NPAD,), jnp.float32)
                       for _ in range(nt)]
        + [pltpu.VMEM((STRIPE,), jnp.float32)],
    )
    def k(*refs):
        t_hbm = refs[:nt]
        ei_hbm = refs[nt]
        o_hbm = refs[nt + 1]
        stab = refs[nt + 2:2 * nt + 2]
        vbuf = refs[-1]

        sid = lax.axis_index("s")
        sl = pl.ds(sid * STRIPE, STRIPE)
        for c in range(nt):
            pltpu.sync_copy(t_hbm[c].at[sl], vbuf)
            pltpu.sync_copy(vbuf, stab[c].at[sl])
        plsc.subcore_barrier()

        def body(di_v, si_v, o_v):
            for c in range(nt):
                pltpu.sync_copy(stab[c].at[di_v.at[0]], o_v.at[c])
                pltpu.sync_copy(stab[c].at[si_v.at[0]], o_v.at[nt + c])

        pltpu.emit_pipeline(
            body,
            grid=(EPAD // W,),
            in_specs=[pl.BlockSpec((1, W), lambda i: (1, i)),
                      pl.BlockSpec((1, W), lambda i: (0, i))],
            out_specs=[pl.BlockSpec((2 * nt, W), lambda i: (0, i))],
            core_axis_name=("c", "s"),
            dimension_semantics=(pltpu.PARALLEL,),
        )(ei_hbm, ei_hbm, o_hbm)

    return k(*tables, ei)


def _sc_scatter(m, ei, npl):
    @functools.partial(
        pl.kernel,
        out_type=jax.ShapeDtypeStruct((2 * npl * NPAD,), jnp.float32),
        mesh=_SC_MESH(),
        scratch_types=[pltpu.VMEM_SHARED((NPAD,), jnp.float32)
                       for _ in range(npl)]
        + [pltpu.VMEM((STRIPE,), jnp.float32)],
    )
    def k(m_hbm, ei_hbm, o_hbm, *scr):
        acc = scr[:npl]
        vbuf = scr[-1]
        cid = lax.axis_index("c")
        sid = lax.axis_index("s")
        sl = pl.ds(sid * STRIPE, STRIPE)

        @pl.loop(0, STRIPE, step=16)
        def _(i):
            vbuf[pl.ds(i, 16)] = jnp.zeros((16,), jnp.float32)

        for c in range(npl):
            pltpu.sync_copy(vbuf, acc[c].at[sl])
        plsc.subcore_barrier()

        def body(m_v, di_v):
            for c in range(npl):
                pltpu.sync_copy(m_v.at[c], acc[c].at[di_v.at[0]], add=True)

        pltpu.emit_pipeline(
            body,
            grid=(EPAD // W,),
            in_specs=[pl.BlockSpec((npl, W), lambda i: (0, i)),
                      pl.BlockSpec((1, W), lambda i: (1, i))],
            out_specs=[],
            core_axis_name=("c", "s"),
            dimension_semantics=(pltpu.PARALLEL,),
        )(m_hbm, ei_hbm)
        plsc.subcore_barrier()
        for c in range(npl):
            pltpu.sync_copy(acc[c].at[sl], vbuf)
            pltpu.sync_copy(
                vbuf,
                o_hbm.at[pl.ds((cid * npl + c) * NPAD + sid * STRIPE,
                               STRIPE)])

    return k(m, ei).reshape(2, npl, NPAD)


def _mlp_body(ni, final_relu, x_ref, waT_ref, wbT_ref, c1_ref,
              w2T_ref, b2_ref, w3T_ref, b3_ref, o_ref):
    x = x_ref[...]
    waT = waT_ref[...]
    wbT = wbT_ref[...]
    acc = None
    for c in range(ni):
        t = (waT[:, c:c + 1] * x[c:c + 1, :]
             + wbT[:, c:c + 1] * x[ni + c:ni + c + 1, :])
        acc = t if acc is None else acc + t
    h = jnp.maximum(acc + c1_ref[...], 0.0)
    h = jnp.maximum(
        jnp.dot(w2T_ref[...], h, precision="highest") + b2_ref[...], 0.0)
    h = jnp.dot(w3T_ref[...], h, precision="highest") + b3_ref[...]
    if final_relu:
        h = jnp.maximum(h, 0.0)
    o_ref[...] = h


def _edge_mlp(xe, waT, wbT, c1, w2T, b2, w3T, b3, final_relu):
    ni = waT.shape[1]
    no = w3T.shape[0]
    full = lambda r, c: pl.BlockSpec((r, c), lambda i: (0, 0))
    return pl.pallas_call(
        functools.partial(_mlp_body, ni, final_relu),
        grid=(EPAD // TILE_E,),
        in_specs=[pl.BlockSpec((2 * ni, TILE_E), lambda i: (0, i)),
                  full(BIG, ni), full(BIG, ni), full(BIG, 1),
                  full(BIG, BIG), full(BIG, 1),
                  full(no, BIG), full(no, 1)],
        out_specs=pl.BlockSpec((no, TILE_E), lambda i: (0, i)),
        out_shape=jax.ShapeDtypeStruct((no, EPAD), jnp.float32),
    )(xe, waT, wbT, c1, w2T, b2, w3T, b3)


def _combine_body(p_ref, h_ref, cnt_ref):
    p = p_ref[...]
    s = p[0] + p[1]
    cnt = s[2:3, :]
    denom = jnp.maximum(cnt, 1.0)
    h_ref[...] = s[0:2, :] / denom
    cnt_ref[...] = cnt


def _combine(penc):
    return pl.pallas_call(
        _combine_body,
        grid=(NPAD // TILE_N,),
        in_specs=[pl.BlockSpec((2, 3, TILE_N), lambda i: (0, 0, i))],
        out_specs=(pl.BlockSpec((2, TILE_N), lambda i: (0, i)),
                   pl.BlockSpec((1, TILE_N), lambda i: (0, i))),
        out_shape=(jax.ShapeDtypeStruct((2, NPAD), jnp.float32),
                   jax.ShapeDtypeStruct((1, NPAD), jnp.float32)),
    )(penc)


def _final_body(p_ref, cnt_ref, x_ref, b_ref, out_ref, emd_ref,
                acc_sum, acc_cnt):
    i = pl.program_id(0)

    @pl.when(i == 0)
    def _():
        acc_sum[...] = jnp.zeros_like(acc_sum)
        acc_cnt[...] = jnp.zeros_like(acc_cnt)

    p = p_ref[...]
    s = p[0] + p[1]
    denom = jnp.maximum(cnt_ref[...], 1.0)
    out = s / denom
    out_ref[...] = out
    d = out - x_ref[...]
    diff = jnp.sum(d * d, axis=0, keepdims=True)
    onehot = (lax.broadcasted_iota(jnp.int32, (G, 1), 0) ==
              b_ref[...]).astype(jnp.float32)
    acc_sum[...] += jnp.sum(onehot * diff, axis=1, keepdims=True)
    acc_cnt[...] += jnp.sum(onehot, axis=1, keepdims=True)

    @pl.when(i == pl.num_programs(0) - 1)
    def _():
        emd_ref[...] = acc_sum[...] / jnp.maximum(acc_cnt[...], 1.0)


def _final(pdec, cnt, xT, batch2):
    return pl.pallas_call(
        _final_body,
        grid=(NPAD // TILE_N,),
        in_specs=[pl.BlockSpec((2, D, TILE_N), lambda i: (0, 0, i)),
                  pl.BlockSpec((1, TILE_N), lambda i: (0, i)),
                  pl.BlockSpec((D, TILE_N), lambda i: (0, i)),
                  pl.BlockSpec((1, TILE_N), lambda i: (0, i))],
        out_specs=(pl.BlockSpec((D, TILE_N), lambda i: (0, i)),
                   pl.BlockSpec((G, 1), lambda i: (0, 0))),
        out_shape=(jax.ShapeDtypeStruct((D, NPAD), jnp.float32),
                   jax.ShapeDtypeStruct((G, 1), jnp.float32)),
        scratch_shapes=[pltpu.VMEM((G, 1), jnp.float32),
                        pltpu.VMEM((G, 1), jnp.float32)],
    )(pdec, cnt, xT, batch2)


def kernel(x, edge_index, batch, bn_gamma, bn_beta,
           enc_w1, enc_b1, enc_w2, enc_b2, enc_w3, enc_b3,
           dec_w1, dec_b1, dec_w2, dec_b2, dec_w3, dec_b3):
    f32 = jnp.float32
    xT = jnp.pad(x.T, ((0, 0), (0, NPAD - N)))
    ei = jnp.pad(edge_index.astype(jnp.int32), ((0, 0), (0, EPAD - E)),
                 constant_values=NPAD - 1)
    batch2 = jnp.pad(batch.astype(jnp.int32), (0, NPAD - N),
                     constant_values=-1).reshape(1, NPAD)

    st = _stats(xT)
    mean = st[:, 0] / N
    var = st[:, 1] / N - mean * mean
    scale = bn_gamma / jnp.sqrt(var + 1e-5)
    shift = bn_beta - mean * scale

    e_waT = (scale[:, None] * (enc_w1[:D] - enc_w1[D:])).T
    e_wbT = (scale[:, None] * enc_w1[D:]).T
    e_c1 = (enc_b1 + shift @ enc_w1[:D]).reshape(BIG, 1)
    e_w3T = jnp.concatenate([enc_w3.T, jnp.zeros((1, BIG), f32)])
    e_b3 = jnp.concatenate([enc_b3, jnp.ones((1,), f32)]).reshape(3, 1)

    d_waT = (dec_w1[:HID] - dec_w1[HID:]).T
    d_wbT = dec_w1[HID:].T
    d_c1 = dec_b1.reshape(BIG, 1)

    xe = _sc_gather([xT[0], xT[1], xT[2], xT[3]], ei)
    m1 = _edge_mlp(xe, e_waT, e_wbT, e_c1, enc_w2.T,
                   enc_b2.reshape(BIG, 1), e_w3T, e_b3, final_relu=True)
    penc = _sc_scatter(m1, ei, 3)
    h2d, cnt = _combine(penc)

    xd = _sc_gather([h2d[0], h2d[1]], ei)
    m2 = _edge_mlp(xd, d_waT, d_wbT, d_c1, dec_w2.T,
                   dec_b2.reshape(BIG, 1), dec_w3.T,
                   dec_b3.reshape(D, 1), final_relu=False)
    pdec = _sc_scatter(m2, ei, 4)

    outT, emd = _final(pdec, cnt, xT, batch2)
    return outT[:, :N].T, emd.reshape(G)
